# Initial kernel scaffold; baseline (speedup 1.0000x reference)
import jax
import jax.numpy as jnp
from jax.experimental import pallas as pl


def kernel(x, nodevec, nodeveck, start_w, start_b, mlp_w, mlp_b, end_w, end_b, gamma, beta):
    raise NotImplementedError("write your pallas kernel here")


# speedup vs baseline: 7.8487x; 7.8487x over previous
import jax
import jax.numpy as jnp
from jax.experimental import pallas as pl
from jax.experimental.pallas import tpu as pltpu

D_MODEL = 256
NODE_DIM = 16
CONV_CH = 16
GDEP = 2
ALPHA = 0.05
TOPK = 4
_INV_SQRT2 = 0.7071067811865476


def _graph_block_kernel(x_ref, nv_ref, nvk_ref, sw_ref, sb_ref, mw_ref,
                        mb_ref, ew_ref, eb_ref, g_ref, b_ref, out_ref):
    f32 = jnp.float32
    N = D_MODEL

    nv1 = nv_ref[...] * nvk_ref[...]
    logits = jax.lax.dot_general(
        nv1, nv1, (((1,), (1,)), ((), ())), preferred_element_type=f32)
    logits = jnp.maximum(logits, 0.0)
    m = jnp.max(logits, axis=1, keepdims=True)
    e = jnp.exp(logits - m)
    adj0 = e / jnp.sum(e, axis=1, keepdims=True)

    col = jax.lax.broadcasted_iota(jnp.int32, (N, N), 1)
    row = jax.lax.broadcasted_iota(jnp.int32, (N, N), 0)
    work = adj0
    mask = jnp.zeros((N, N), f32)
    for _ in range(TOPK):
        mx = jnp.max(work, axis=1, keepdims=True)
        first = jnp.min(jnp.where(work == mx, col, N), axis=1, keepdims=True)
        sel = col == first
        mask = jnp.where(sel, 1.0, mask)
        work = jnp.where(sel, -jnp.inf, work)

    adj = adj0 * mask + jnp.where(row == col, 1.0, 0.0)
    d = jnp.sum(adj, axis=1, keepdims=True)
    aw = adj * ((1.0 - ALPHA) / d)

    X = x_ref[...]
    nt = (((1,), (1,)), ((), ()))
    Y1 = ALPHA * X + jax.lax.dot_general(X, aw, nt, preferred_element_type=f32)
    Y2 = ALPHA * X + jax.lax.dot_general(Y1, aw, nt, preferred_element_type=f32)

    s = sw_ref[...]
    sb = sb_ref[...]
    W = mw_ref[...]
    s3 = jnp.concatenate([s, s, s], axis=1)
    b3 = jnp.concatenate([sb, sb, sb], axis=1)
    ws = W * s3
    w0 = jnp.sum(ws[:, 0:CONV_CH], axis=1, keepdims=True)
    w1 = jnp.sum(ws[:, CONV_CH:2 * CONV_CH], axis=1, keepdims=True)
    w2 = jnp.sum(ws[:, 2 * CONV_CH:3 * CONV_CH], axis=1, keepdims=True)
    cst = jnp.sum(W * b3, axis=1, keepdims=True) + mb_ref[...].T
    ew = ew_ref[...]

    acc = None
    for o in range(CONV_CH):
        u = (w0[o:o + 1, :] * X + w1[o:o + 1, :] * Y1
             + w2[o:o + 1, :] * Y2 + cst[o:o + 1, :])
        g = 0.5 * u * (1.0 + jax.lax.erf(u * _INV_SQRT2))
        t = ew[:, o:o + 1] * g
        acc = t if acc is None else acc + t

    y = X + acc + eb_ref[...]
    mu = jnp.mean(y, axis=1, keepdims=True)
    yc = y - mu
    var = jnp.mean(yc * yc, axis=1, keepdims=True)
    out_ref[...] = yc / jnp.sqrt(var + 1e-5) * g_ref[...] + b_ref[...]


def kernel(x, nodevec, nodeveck, start_w, start_b, mlp_w, mlp_b, end_w,
           end_b, gamma, beta):
    B, T, D = x.shape
    xr = x.reshape(B * T, D)
    out = pl.pallas_call(
        _graph_block_kernel,
        out_shape=jax.ShapeDtypeStruct((B * T, D), jnp.float32),
        compiler_params=pltpu.CompilerParams(
            vmem_limit_bytes=100 * 1024 * 1024),
    )(xr, nodevec, nodeveck,
      start_w.reshape(1, CONV_CH), start_b.reshape(1, CONV_CH),
      mlp_w, mlp_b.reshape(1, CONV_CH),
      end_w.reshape(1, CONV_CH), end_b.reshape(1, 1),
      gamma.reshape(1, D), beta.reshape(1, D))
    return out.reshape(B, T, D)


# speedup vs baseline: 8.9640x; 1.1421x over previous
import jax
import jax.numpy as jnp
from jax.experimental import pallas as pl
from jax.experimental.pallas import tpu as pltpu

D_MODEL = 256
CONV_CH = 16
ALPHA = 0.05
TOPK = 4
GRID = 4
_SQRT2 = 1.4142135623730951
_INV_SQRT2 = 0.7071067811865476


def _graph_block_kernel(x_ref, nv_ref, nvk_ref, sw_ref, sb_ref, mw_ref,
                        mb_ref, ew_ref, eb_ref, g_ref, b_ref, out_ref,
                        aw_ref):
    f32 = jnp.float32
    N = D_MODEL

    @pl.when(pl.program_id(0) == 0)
    def _build_adjacency():
        nv1 = nv_ref[...] * nvk_ref[...]
        logits = jax.lax.dot_general(
            nv1, nv1, (((1,), (1,)), ((), ())), preferred_element_type=f32)
        logits = jnp.maximum(logits, 0.0)
        m = jnp.max(logits, axis=1, keepdims=True)
        e = jnp.exp(logits - m)
        adj0 = e / jnp.sum(e, axis=1, keepdims=True)

        work = adj0
        for _ in range(TOPK - 1):
            mx = jnp.max(work, axis=1, keepdims=True)
            work = jnp.where(work == mx, -jnp.inf, work)
        thresh = jnp.max(work, axis=1, keepdims=True)
        keep = adj0 >= thresh

        row = jax.lax.broadcasted_iota(jnp.int32, (N, N), 0)
        col = jax.lax.broadcasted_iota(jnp.int32, (N, N), 1)
        adj = jnp.where(keep, adj0, 0.0) + jnp.where(row == col, 1.0, 0.0)
        d = jnp.sum(adj, axis=1, keepdims=True)
        aw_ref[...] = adj * ((1.0 - ALPHA) / d)

    X = x_ref[...]
    aw = aw_ref[...]
    nt = (((1,), (1,)), ((), ()))
    Y1 = ALPHA * X + jax.lax.dot_general(X, aw, nt, preferred_element_type=f32)
    Y2 = ALPHA * X + jax.lax.dot_general(Y1, aw, nt, preferred_element_type=f32)

    s = sw_ref[...]
    sb = sb_ref[...]
    W = mw_ref[...]
    s3 = jnp.concatenate([s, s, s], axis=1) * _INV_SQRT2
    b3 = jnp.concatenate([sb, sb, sb], axis=1) * _INV_SQRT2
    ws = W * s3
    w0 = jnp.sum(ws[:, 0:CONV_CH], axis=1, keepdims=True)
    w1 = jnp.sum(ws[:, CONV_CH:2 * CONV_CH], axis=1, keepdims=True)
    w2 = jnp.sum(ws[:, 2 * CONV_CH:3 * CONV_CH], axis=1, keepdims=True)
    cst = (jnp.sum(W * b3, axis=1, keepdims=True)
           + mb_ref[...].T * _INV_SQRT2)
    q = ew_ref[...].T * _INV_SQRT2

    p0 = jnp.sum(q * w0, keepdims=True)
    p1 = jnp.sum(q * w1, keepdims=True)
    p2 = jnp.sum(q * w2, keepdims=True)
    pc = jnp.sum(q * cst, keepdims=True)

    acc = p0 * X + p1 * Y1 + p2 * Y2 + (pc + eb_ref[...])
    for o in range(CONV_CH):
        u = (w0[o:o + 1, :] * X + w1[o:o + 1, :] * Y1
             + w2[o:o + 1, :] * Y2 + cst[o:o + 1, :])
        acc = acc + q[o:o + 1, :] * (u * jax.lax.erf(u))

    y = X + acc
    mu = jnp.mean(y, axis=1, keepdims=True)
    yc = y - mu
    var = jnp.mean(yc * yc, axis=1, keepdims=True)
    out_ref[...] = yc / jnp.sqrt(var + 1e-5) * g_ref[...] + b_ref[...]


def kernel(x, nodevec, nodeveck, start_w, start_b, mlp_w, mlp_b, end_w,
           end_b, gamma, beta):
    B, T, D = x.shape
    R = B * T
    blk = R // GRID
    xr = x.reshape(R, D)
    full = lambda i: (0, 0)
    out = pl.pallas_call(
        _graph_block_kernel,
        grid=(GRID,),
        in_specs=[
            pl.BlockSpec((blk, D), lambda i: (i, 0)),
            pl.BlockSpec((D, CONV_CH), full),
            pl.BlockSpec((1, CONV_CH), full),
            pl.BlockSpec((1, CONV_CH), full),
            pl.BlockSpec((1, CONV_CH), full),
            pl.BlockSpec((CONV_CH, 3 * CONV_CH), full),
            pl.BlockSpec((1, CONV_CH), full),
            pl.BlockSpec((1, CONV_CH), full),
            pl.BlockSpec((1, 1), full),
            pl.BlockSpec((1, D), full),
            pl.BlockSpec((1, D), full),
        ],
        out_specs=pl.BlockSpec((blk, D), lambda i: (i, 0)),
        out_shape=jax.ShapeDtypeStruct((R, D), jnp.float32),
        scratch_shapes=[pltpu.VMEM((D, D), jnp.float32)],
        compiler_params=pltpu.CompilerParams(
            vmem_limit_bytes=100 * 1024 * 1024),
    )(xr, nodevec, nodeveck,
      start_w.reshape(1, CONV_CH), start_b.reshape(1, CONV_CH),
      mlp_w, mlp_b.reshape(1, CONV_CH),
      end_w.reshape(1, CONV_CH), end_b.reshape(1, 1),
      gamma.reshape(1, D), beta.reshape(1, D))
    return out.reshape(B, T, D)


# speedup vs baseline: 10.2086x; 1.1388x over previous
import jax
import jax.numpy as jnp
from jax.experimental import pallas as pl
from jax.experimental.pallas import tpu as pltpu

D_MODEL = 256
CONV_CH = 16
ALPHA = 0.05
TOPK = 4
GRID = 4
_INV_SQRT2 = 0.7071067811865476


def _graph_block_kernel(x_ref, nv_ref, nvk_ref, sw_ref, sb_ref, mw_ref,
                        mb_ref, ew_ref, eb_ref, g_ref, b_ref, out_ref,
                        m_ref, f_ref):
    f32 = jnp.float32
    N = D_MODEL
    nn = (((1,), (0,)), ((), ()))

    @pl.when(pl.program_id(0) == 0)
    def _precompute():
        nv1 = nv_ref[...] * nvk_ref[...]
        logits = jax.lax.dot_general(
            nv1, nv1, (((1,), (1,)), ((), ())), preferred_element_type=f32)
        logits = jnp.maximum(logits, 0.0)
        m = jnp.max(logits, axis=0, keepdims=True)
        e = jnp.exp(logits - m)
        adj0t = e / jnp.sum(e, axis=0, keepdims=True)

        work = adj0t
        for _ in range(TOPK - 1):
            mx = jnp.max(work, axis=0, keepdims=True)
            work = jnp.where(work == mx, -jnp.inf, work)
        thresh = jnp.max(work, axis=0, keepdims=True)
        keep = adj0t >= thresh

        row = jax.lax.broadcasted_iota(jnp.int32, (N, N), 0)
        col = jax.lax.broadcasted_iota(jnp.int32, (N, N), 1)
        adjt = jnp.where(keep, adj0t, 0.0) + jnp.where(row == col, 1.0, 0.0)
        d = jnp.sum(adjt, axis=0, keepdims=True)
        P = adjt * ((1.0 - ALPHA) / d)
        P2 = jax.lax.dot_general(P, P, nn, preferred_element_type=f32)

        s = sw_ref[...]
        sb = sb_ref[...]
        W = mw_ref[...]
        s3 = jnp.concatenate([s, s, s], axis=1) * _INV_SQRT2
        b3 = jnp.concatenate([sb, sb, sb], axis=1) * _INV_SQRT2
        ws = W * s3
        w0 = jnp.sum(ws[:, 0:CONV_CH], axis=1, keepdims=True)
        w1 = jnp.sum(ws[:, CONV_CH:2 * CONV_CH], axis=1, keepdims=True)
        w2 = jnp.sum(ws[:, 2 * CONV_CH:3 * CONV_CH], axis=1, keepdims=True)
        cst = (jnp.sum(W * b3, axis=1, keepdims=True)
               + mb_ref[...].T * _INV_SQRT2)
        q = ew_ref[...].T * _INV_SQRT2
        f_ref[0:1, :] = cst.T
        f_ref[1:2, :] = q.T

        b1 = w1 + ALPHA * w2
        b0 = w0 + ALPHA * w1 + ALPHA * w2
        eye = jnp.where(row == col, 1.0, 0.0)
        for o in range(CONV_CH):
            m_ref[:, o * N:(o + 1) * N] = (
                b0[o:o + 1, 0:1] * eye + b1[o:o + 1, 0:1] * P
                + w2[o:o + 1, 0:1] * P2)

    X = x_ref[...]
    acc = jnp.zeros_like(X) + eb_ref[...]
    for o in range(CONV_CH):
        u = jax.lax.dot_general(X, m_ref[:, o * N:(o + 1) * N], nn,
                                preferred_element_type=f32)
        u = u + f_ref[0:1, o:o + 1]
        acc = acc + f_ref[1:2, o:o + 1] * (u * (1.0 + jax.lax.erf(u)))

    y = X + acc
    mu = jnp.mean(y, axis=1, keepdims=True)
    yc = y - mu
    var = jnp.mean(yc * yc, axis=1, keepdims=True)
    out_ref[...] = yc / jnp.sqrt(var + 1e-5) * g_ref[...] + b_ref[...]


def kernel(x, nodevec, nodeveck, start_w, start_b, mlp_w, mlp_b, end_w,
           end_b, gamma, beta):
    B, T, D = x.shape
    R = B * T
    blk = R // GRID
    xr = x.reshape(R, D)
    full = lambda i: (0, 0)
    out = pl.pallas_call(
        _graph_block_kernel,
        grid=(GRID,),
        in_specs=[
            pl.BlockSpec((blk, D), lambda i: (i, 0)),
            pl.BlockSpec((D, CONV_CH), full),
            pl.BlockSpec((1, CONV_CH), full),
            pl.BlockSpec((1, CONV_CH), full),
            pl.BlockSpec((1, CONV_CH), full),
            pl.BlockSpec((CONV_CH, 3 * CONV_CH), full),
            pl.BlockSpec((1, CONV_CH), full),
            pl.BlockSpec((1, CONV_CH), full),
            pl.BlockSpec((1, 1), full),
            pl.BlockSpec((1, D), full),
            pl.BlockSpec((1, D), full),
        ],
        out_specs=pl.BlockSpec((blk, D), lambda i: (i, 0)),
        out_shape=jax.ShapeDtypeStruct((R, D), jnp.float32),
        scratch_shapes=[pltpu.VMEM((D, CONV_CH * D), jnp.float32),
                        pltpu.VMEM((2, CONV_CH), jnp.float32)],
        compiler_params=pltpu.CompilerParams(
            vmem_limit_bytes=100 * 1024 * 1024),
    )(xr, nodevec, nodeveck,
      start_w.reshape(1, CONV_CH), start_b.reshape(1, CONV_CH),
      mlp_w, mlp_b.reshape(1, CONV_CH),
      end_w.reshape(1, CONV_CH), end_b.reshape(1, 1),
      gamma.reshape(1, D), beta.reshape(1, D))
    return out.reshape(B, T, D)


# speedup vs baseline: 10.8099x; 1.0589x over previous
import jax
import jax.numpy as jnp
from jax.experimental import pallas as pl
from jax.experimental.pallas import tpu as pltpu

D_MODEL = 256
CONV_CH = 16
ALPHA = 0.05
TOPK = 4
GRID = 4
_INV_SQRT2 = 0.7071067811865476


def _graph_block_kernel(x_ref, nv_ref, nvk_ref, sw_ref, sb_ref, mw_ref,
                        mb_ref, ew_ref, eb_ref, g_ref, b_ref, out_ref,
                        m_ref, f_ref):
    f32 = jnp.float32
    N = D_MODEL
    nn = (((1,), (0,)), ((), ()))

    @pl.when(pl.program_id(0) == 0)
    def _precompute():
        nv1 = nv_ref[...] * nvk_ref[...]
        logits = jax.lax.dot_general(
            nv1, nv1, (((1,), (1,)), ((), ())), preferred_element_type=f32)
        logits = jnp.maximum(logits, 0.0)
        m = jnp.max(logits, axis=0, keepdims=True)
        e = jnp.exp(logits - m)
        adj0t = e / jnp.sum(e, axis=0, keepdims=True)

        work = adj0t
        for _ in range(TOPK - 1):
            mx = jnp.max(work, axis=0, keepdims=True)
            work = jnp.where(work == mx, -jnp.inf, work)
        thresh = jnp.max(work, axis=0, keepdims=True)
        keep = adj0t >= thresh

        row = jax.lax.broadcasted_iota(jnp.int32, (N, N), 0)
        col = jax.lax.broadcasted_iota(jnp.int32, (N, N), 1)
        adjt = jnp.where(keep, adj0t, 0.0) + jnp.where(row == col, 1.0, 0.0)
        d = jnp.sum(adjt, axis=0, keepdims=True)
        P = adjt * ((1.0 - ALPHA) / d)
        P2 = jax.lax.dot_general(P, P, nn, preferred_element_type=f32)

        s = sw_ref[...]
        sb = sb_ref[...]
        W = mw_ref[...]
        s3 = jnp.concatenate([s, s, s], axis=1) * _INV_SQRT2
        b3 = jnp.concatenate([sb, sb, sb], axis=1) * _INV_SQRT2
        ws = W * s3
        w0 = jnp.sum(ws[:, 0:CONV_CH], axis=1, keepdims=True)
        w1 = jnp.sum(ws[:, CONV_CH:2 * CONV_CH], axis=1, keepdims=True)
        w2 = jnp.sum(ws[:, 2 * CONV_CH:3 * CONV_CH], axis=1, keepdims=True)
        cst = (jnp.sum(W * b3, axis=1, keepdims=True)
               + mb_ref[...].T * _INV_SQRT2)
        q = ew_ref[...].T * _INV_SQRT2
        f_ref[0:1, :] = cst.T
        f_ref[1:2, :] = q.T
        f_ref[2:3, 0:1] = jnp.sum(q * cst, keepdims=True) + eb_ref[...]

        b1 = w1 + ALPHA * w2
        b0 = w0 + ALPHA * w1 + ALPHA * w2
        eye = jnp.where(row == col, 1.0, 0.0)
        ml = jnp.zeros((N, N), f32)
        for o in range(CONV_CH):
            mo = (b0[o:o + 1, 0:1] * eye + b1[o:o + 1, 0:1] * P
                  + w2[o:o + 1, 0:1] * P2)
            ml = ml + q[o:o + 1, 0:1] * mo
            m_ref[:, o * N:(o + 1) * N] = mo.astype(jnp.bfloat16)
        m_ref[:, CONV_CH * N:(CONV_CH + 1) * N] = ml.astype(jnp.bfloat16)

    X = x_ref[...]
    Xb = X.astype(jnp.bfloat16)
    acc = jax.lax.dot_general(Xb, m_ref[:, CONV_CH * N:(CONV_CH + 1) * N],
                              nn, preferred_element_type=f32)
    acc = acc + f_ref[2:3, 0:1]
    for o in range(CONV_CH):
        u = jax.lax.dot_general(Xb, m_ref[:, o * N:(o + 1) * N], nn,
                                preferred_element_type=f32)
        u = u + f_ref[0:1, o:o + 1]
        acc = acc + f_ref[1:2, o:o + 1] * (u * jax.lax.erf(u))

    y = X + acc
    mu = jnp.mean(y, axis=1, keepdims=True)
    yc = y - mu
    var = jnp.mean(yc * yc, axis=1, keepdims=True)
    out_ref[...] = yc / jnp.sqrt(var + 1e-5) * g_ref[...] + b_ref[...]


def kernel(x, nodevec, nodeveck, start_w, start_b, mlp_w, mlp_b, end_w,
           end_b, gamma, beta):
    B, T, D = x.shape
    R = B * T
    blk = R // GRID
    xr = x.reshape(R, D)
    full = lambda i: (0, 0)
    out = pl.pallas_call(
        _graph_block_kernel,
        grid=(GRID,),
        in_specs=[
            pl.BlockSpec((blk, D), lambda i: (i, 0)),
            pl.BlockSpec((D, CONV_CH), full),
            pl.BlockSpec((1, CONV_CH), full),
            pl.BlockSpec((1, CONV_CH), full),
            pl.BlockSpec((1, CONV_CH), full),
            pl.BlockSpec((CONV_CH, 3 * CONV_CH), full),
            pl.BlockSpec((1, CONV_CH), full),
            pl.BlockSpec((1, CONV_CH), full),
            pl.BlockSpec((1, 1), full),
            pl.BlockSpec((1, D), full),
            pl.BlockSpec((1, D), full),
        ],
        out_specs=pl.BlockSpec((blk, D), lambda i: (i, 0)),
        out_shape=jax.ShapeDtypeStruct((R, D), jnp.float32),
        scratch_shapes=[pltpu.VMEM((D, (CONV_CH + 1) * D), jnp.bfloat16),
                        pltpu.VMEM((3, CONV_CH), jnp.float32)],
        compiler_params=pltpu.CompilerParams(
            vmem_limit_bytes=100 * 1024 * 1024),
    )(xr, nodevec, nodeveck,
      start_w.reshape(1, CONV_CH), start_b.reshape(1, CONV_CH),
      mlp_w, mlp_b.reshape(1, CONV_CH),
      end_w.reshape(1, CONV_CH), end_b.reshape(1, 1),
      gamma.reshape(1, D), beta.reshape(1, D))
    return out.reshape(B, T, D)


# speedup vs baseline: 11.5713x; 1.0704x over previous
import jax
import jax.numpy as jnp
from jax.experimental import pallas as pl
from jax.experimental.pallas import tpu as pltpu

D_MODEL = 256
CONV_CH = 16
ALPHA = 0.05
TOPK = 4
GRID = 1
_INV_SQRT2 = 0.7071067811865476


def _graph_block_kernel(x_ref, nv_ref, nvk_ref, sw_ref, sb_ref, mw_ref,
                        mb_ref, ew_ref, eb_ref, g_ref, b_ref, out_ref,
                        m_ref, f_ref):
    f32 = jnp.float32
    N = D_MODEL
    nn = (((1,), (0,)), ((), ()))

    @pl.when(pl.program_id(0) == 0)
    def _precompute():
        nv1 = nv_ref[...] * nvk_ref[...]
        logits = jax.lax.dot_general(
            nv1, nv1, (((1,), (1,)), ((), ())), preferred_element_type=f32)
        logits = jnp.maximum(logits, 0.0)
        m = jnp.max(logits, axis=0, keepdims=True)
        e = jnp.exp(logits - m)
        adj0t = e / jnp.sum(e, axis=0, keepdims=True)

        work = adj0t
        for _ in range(TOPK - 1):
            mx = jnp.max(work, axis=0, keepdims=True)
            work = jnp.where(work == mx, -jnp.inf, work)
        thresh = jnp.max(work, axis=0, keepdims=True)
        keep = adj0t >= thresh

        row = jax.lax.broadcasted_iota(jnp.int32, (N, N), 0)
        col = jax.lax.broadcasted_iota(jnp.int32, (N, N), 1)
        adjt = jnp.where(keep, adj0t, 0.0) + jnp.where(row == col, 1.0, 0.0)
        d = jnp.sum(adjt, axis=0, keepdims=True)
        P = adjt * ((1.0 - ALPHA) / d)
        P2 = jax.lax.dot_general(P, P, nn, preferred_element_type=f32)

        s = sw_ref[...]
        sb = sb_ref[...]
        W = mw_ref[...]
        s3 = jnp.concatenate([s, s, s], axis=1) * _INV_SQRT2
        b3 = jnp.concatenate([sb, sb, sb], axis=1) * _INV_SQRT2
        ws = W * s3
        w0 = jnp.sum(ws[:, 0:CONV_CH], axis=1, keepdims=True)
        w1 = jnp.sum(ws[:, CONV_CH:2 * CONV_CH], axis=1, keepdims=True)
        w2 = jnp.sum(ws[:, 2 * CONV_CH:3 * CONV_CH], axis=1, keepdims=True)
        cst = (jnp.sum(W * b3, axis=1, keepdims=True)
               + mb_ref[...].T * _INV_SQRT2)
        q = ew_ref[...].T * _INV_SQRT2
        f_ref[0:1, :] = cst.T
        f_ref[1:2, :] = q.T
        f_ref[2:3, 0:1] = jnp.sum(q * cst, keepdims=True) + eb_ref[...]

        b1 = w1 + ALPHA * w2
        b0 = w0 + ALPHA * w1 + ALPHA * w2
        eye = jnp.where(row == col, 1.0, 0.0)
        ml = jnp.zeros((N, N), f32)
        for o in range(CONV_CH):
            mo = (b0[o:o + 1, 0:1] * eye + b1[o:o + 1, 0:1] * P
                  + w2[o:o + 1, 0:1] * P2)
            ml = ml + q[o:o + 1, 0:1] * mo
            m_ref[:, o * N:(o + 1) * N] = mo.astype(jnp.bfloat16)
        m_ref[:, CONV_CH * N:(CONV_CH + 1) * N] = ml.astype(jnp.bfloat16)

    X = x_ref[...]
    Xb = X.astype(jnp.bfloat16)
    acc = jax.lax.dot_general(Xb, m_ref[:, CONV_CH * N:(CONV_CH + 1) * N],
                              nn, preferred_element_type=f32)
    acc = acc + f_ref[2:3, 0:1]
    for o in range(CONV_CH):
        u = jax.lax.dot_general(Xb, m_ref[:, o * N:(o + 1) * N], nn,
                                preferred_element_type=f32)
        u = u + f_ref[0:1, o:o + 1]
        acc = acc + f_ref[1:2, o:o + 1] * (u * jax.lax.erf(u))

    y = X + acc
    mu = jnp.mean(y, axis=1, keepdims=True)
    yc = y - mu
    var = jnp.mean(yc * yc, axis=1, keepdims=True)
    out_ref[...] = yc / jnp.sqrt(var + 1e-5) * g_ref[...] + b_ref[...]


def kernel(x, nodevec, nodeveck, start_w, start_b, mlp_w, mlp_b, end_w,
           end_b, gamma, beta):
    B, T, D = x.shape
    R = B * T
    blk = R // GRID
    xr = x.reshape(R, D)
    full = lambda i: (0, 0)
    out = pl.pallas_call(
        _graph_block_kernel,
        grid=(GRID,),
        in_specs=[
            pl.BlockSpec((blk, D), lambda i: (i, 0)),
            pl.BlockSpec((D, CONV_CH), full),
            pl.BlockSpec((1, CONV_CH), full),
            pl.BlockSpec((1, CONV_CH), full),
            pl.BlockSpec((1, CONV_CH), full),
            pl.BlockSpec((CONV_CH, 3 * CONV_CH), full),
            pl.BlockSpec((1, CONV_CH), full),
            pl.BlockSpec((1, CONV_CH), full),
            pl.BlockSpec((1, 1), full),
            pl.BlockSpec((1, D), full),
            pl.BlockSpec((1, D), full),
        ],
        out_specs=pl.BlockSpec((blk, D), lambda i: (i, 0)),
        out_shape=jax.ShapeDtypeStruct((R, D), jnp.float32),
        scratch_shapes=[pltpu.VMEM((D, (CONV_CH + 1) * D), jnp.bfloat16),
                        pltpu.VMEM((3, CONV_CH), jnp.float32)],
        compiler_params=pltpu.CompilerParams(
            vmem_limit_bytes=100 * 1024 * 1024),
    )(xr, nodevec, nodeveck,
      start_w.reshape(1, CONV_CH), start_b.reshape(1, CONV_CH),
      mlp_w, mlp_b.reshape(1, CONV_CH),
      end_w.reshape(1, CONV_CH), end_b.reshape(1, 1),
      gamma.reshape(1, D), beta.reshape(1, D))
    return out.reshape(B, T, D)
